# baseline (speedup 1.0000x reference)
import functools

import jax
import jax.numpy as jnp
from jax import lax
from jax.experimental import pallas as pl
from jax.experimental.pallas import tpu as pltpu
from jax.experimental.pallas import tpu_sc as plsc

N = 16384
C = 1000
NC = 2
NS = 16
NW = NC * NS
TB = N // NW
CB = 40
C0B = 8
CTB = 32
L = 16


def _onehot_t_body(x_hbm, out_hbm, buf0, buf1, xbuf, sem0, sem1):
    wid = lax.axis_index("s") * NC + lax.axis_index("c")
    col0 = pl.multiple_of(wid * TB, 128)

    pltpu.async_copy(x_hbm.at[pl.ds(wid * TB, TB)], xbuf, sem0)

    zeros = jnp.zeros((L,), jnp.int32)
    ones = jnp.full((L,), 1, jnp.int32)
    iota = lax.iota(jnp.int32, L)

    def zero_rows(buf, r0, r1):
        def _zero(r, carry):
            for j in range(TB // L):
                buf[r, pl.ds(j * L, L)] = zeros
            return carry

        lax.fori_loop(r0, r1, _zero, 0)

    def scatter_chunk(buf, c0, vals, cb):
        def _group(g, carry):
            b0 = g * L
            xv = xbuf[pl.ds(b0, L)]
            rows = xv - c0
            mask = plsc.bitcast(rows, jnp.uint32) < jnp.uint32(cb)
            plsc.store_scatter(buf, [rows, iota + b0], vals, mask=mask)
            return carry

        lax.fori_loop(0, TB // L, _group, 0)

    def fire(buf, c0, sem, cb):
        pltpu.async_copy(
            buf.at[pl.ds(0, cb)], out_hbm.at[pl.ds(c0, cb), pl.ds(col0, TB)], sem
        )

    def drain(buf, sem, cb):
        pltpu.make_async_copy(
            buf.at[pl.ds(0, cb)], out_hbm.at[pl.ds(0, cb), pl.ds(col0, TB)], sem
        ).wait()

    zero_rows(buf0, 0, C0B)
    pltpu.make_async_copy(x_hbm.at[pl.ds(wid * TB, TB)], xbuf, sem0).wait()
    scatter_chunk(buf0, 0, ones, C0B)
    fire(buf0, 0, sem0, C0B)

    zero_rows(buf1, 0, CB)
    scatter_chunk(buf1, C0B, ones, CB)
    fire(buf1, C0B, sem1, CB)
    zero_rows(buf0, C0B, CB)

    drain(buf0, sem0, C0B)
    scatter_chunk(buf0, 0, zeros, C0B)
    scatter_chunk(buf0, C0B + CB, ones, CB)
    fire(buf0, C0B + CB, sem0, CB)

    def _pair(p, carry):
        for b, (buf, sem) in enumerate(((buf1, sem1), (buf0, sem0))):
            c0 = pl.multiple_of(C0B + (2 * p + 2 + b) * CB, 8)
            drain(buf, sem, CB)
            scatter_chunk(buf, c0 - 2 * CB, zeros, CB)
            scatter_chunk(buf, c0, ones, CB)
            fire(buf, c0, sem, CB)
        return carry

    lax.fori_loop(0, 11, _pair, 0)

    drain(buf1, sem1, CB)
    scatter_chunk(buf1, C0B + 22 * CB, zeros, CB)
    scatter_chunk(buf1, C0B + 24 * CB, ones, CTB)
    fire(buf1, C0B + 24 * CB, sem1, CTB)
    drain(buf0, sem0, CB)
    drain(buf1, sem1, CTB)


@functools.partial(
    pl.kernel,
    out_type=jax.ShapeDtypeStruct((C, N), jnp.int32),
    mesh=plsc.VectorSubcoreMesh(
        core_axis_name="c", subcore_axis_name="s", num_cores=NC, num_subcores=NS
    ),
    scratch_types=[
        pltpu.VMEM((CB, TB), jnp.int32),
        pltpu.VMEM((CB, TB), jnp.int32),
        pltpu.VMEM((TB,), jnp.int32),
        pltpu.SemaphoreType.DMA,
        pltpu.SemaphoreType.DMA,
    ],
    compiler_params=pltpu.CompilerParams(needs_layout_passes=False),
)
def _sc_onehot_t(x_hbm, out_hbm, buf0, buf1, xbuf, sem0, sem1):
    _onehot_t_body(x_hbm, out_hbm, buf0, buf1, xbuf, sem0, sem1)


def kernel(x):
    return _sc_onehot_t(x).T


# speedup vs baseline: 1.0408x; 1.0408x over previous
import functools

import jax
import jax.numpy as jnp
from jax import lax
from jax.experimental import pallas as pl
from jax.experimental.pallas import tpu as pltpu
from jax.experimental.pallas import tpu_sc as plsc

N = 16384
C = 1000
NC = 2
NS = 16
NW = NC * NS
TB = N // NW
CB = 40
NCH = C // CB
L = 16


def _onehot_t_body(x_hbm, out_hbm, buf0, buf1, xbuf, sem0, sem1):
    wid = lax.axis_index("s") * NC + lax.axis_index("c")
    col0 = pl.multiple_of(wid * TB, 128)

    pltpu.sync_copy(x_hbm.at[pl.ds(wid * TB, TB)], xbuf)

    zeros = jnp.zeros((L,), jnp.int32)
    ones = jnp.full((L,), 1, jnp.int32)
    iota = lax.iota(jnp.int32, L)

    def _zero_rows(buf):
        def _zero(r, carry):
            for j in range(TB // L):
                buf[r, pl.ds(j * L, L)] = zeros
            return carry

        lax.fori_loop(0, CB, _zero, 0)

    def scatter_chunk(buf, c0, vals):
        def _group(g, carry):
            b0 = g * L
            xv = xbuf[pl.ds(b0, L)]
            rows = xv - c0
            mask = plsc.bitcast(rows, jnp.uint32) < jnp.uint32(CB)
            plsc.store_scatter(buf, [rows, iota + b0], vals, mask=mask)
            return carry

        lax.fori_loop(0, TB // L, _group, 0)

    def fire(buf, c0, sem):
        pltpu.async_copy(
            buf, out_hbm.at[pl.ds(c0, CB), pl.ds(col0, TB)], sem
        )

    def drain(buf, sem):
        pltpu.make_async_copy(
            buf, out_hbm.at[pl.ds(0, CB), pl.ds(col0, TB)], sem
        ).wait()

    _zero_rows(buf0)
    scatter_chunk(buf0, 0, ones)
    fire(buf0, 0, sem0)
    _zero_rows(buf1)
    scatter_chunk(buf1, CB, ones)
    fire(buf1, CB, sem1)

    def _pair(p, carry):
        for b, (buf, sem) in enumerate(((buf0, sem0), (buf1, sem1))):
            c0 = pl.multiple_of((2 * p + b) * CB, 8)
            drain(buf, sem)
            scatter_chunk(buf, c0 - 2 * CB, zeros)
            scatter_chunk(buf, c0, ones)
            fire(buf, c0, sem)
        return carry

    lax.fori_loop(1, (NCH - 1) // 2, _pair, 0)

    drain(buf0, sem0)
    scatter_chunk(buf0, (NCH - 3) * CB, zeros)
    scatter_chunk(buf0, (NCH - 1) * CB, ones)
    fire(buf0, (NCH - 1) * CB, sem0)
    drain(buf1, sem1)
    drain(buf0, sem0)


@functools.partial(
    pl.kernel,
    out_type=jax.ShapeDtypeStruct((C, N), jnp.int32),
    mesh=plsc.VectorSubcoreMesh(
        core_axis_name="c", subcore_axis_name="s", num_cores=NC, num_subcores=NS
    ),
    scratch_types=[
        pltpu.VMEM((CB, TB), jnp.int32),
        pltpu.VMEM((CB, TB), jnp.int32),
        pltpu.VMEM((TB,), jnp.int32),
        pltpu.SemaphoreType.DMA,
        pltpu.SemaphoreType.DMA,
    ],
    compiler_params=pltpu.CompilerParams(needs_layout_passes=False),
)
def _sc_onehot_t(x_hbm, out_hbm, buf0, buf1, xbuf, sem0, sem1):
    _onehot_t_body(x_hbm, out_hbm, buf0, buf1, xbuf, sem0, sem1)


def kernel(x):
    return _sc_onehot_t(x).T
